# baseline (speedup 1.0000x reference)
import jax
import jax.numpy as jnp
from jax.experimental import pallas as pl
from jax.experimental.pallas import tpu as pltpu

B = 16384
D_IN = 1024
D_HID = 512
D_OUT = 512

CH = 1024
NCHUNK = B // CH
NX = 10
NO = 6


def _mlp_kernel(x_hbm, w1_ref, b1_ref, w2_ref, b2_ref, o_hbm,
                x_bufs, o_bufs, in_sems, out_sems):
    w1 = w1_ref[:]
    b1 = b1_ref[:]
    w2 = w2_ref[:]
    b2 = b2_ref[:]

    def in_copy(i, slot):
        return pltpu.make_async_copy(
            x_hbm.at[pl.ds(i * CH, CH), :], x_bufs.at[slot], in_sems.at[slot])

    def out_copy(i, slot):
        return pltpu.make_async_copy(
            o_bufs.at[slot], o_hbm.at[pl.ds(i * CH, CH), :], out_sems.at[slot])

    for k in range(NX):
        in_copy(k, k).start()

    for i in range(NCHUNK):
        slot = i % NX
        oslot = i % NO
        in_copy(i, slot).wait()

        h = jnp.dot(x_bufs[slot], w1, preferred_element_type=jnp.float32)
        h = jnp.maximum(h + b1, 0.0)
        o = jnp.dot(h, w2, preferred_element_type=jnp.float32) + b2

        if i >= NO:
            out_copy(i - NO, oslot).wait()

        o_bufs[oslot] = o
        out_copy(i, oslot).start()

        if i + NX < NCHUNK:
            in_copy(i + NX, slot).start()

    for k in range(NO):
        i = NCHUNK - NO + k
        out_copy(i, i % NO).wait()


@jax.jit
def kernel(image, W1, b1, W2, b2):
    b1r = b1.reshape(1, D_HID)
    b2r = b2.reshape(1, D_OUT)
    return pl.pallas_call(
        _mlp_kernel,
        in_specs=[
            pl.BlockSpec(memory_space=pltpu.MemorySpace.HBM),
            pl.BlockSpec(memory_space=pltpu.MemorySpace.VMEM),
            pl.BlockSpec(memory_space=pltpu.MemorySpace.VMEM),
            pl.BlockSpec(memory_space=pltpu.MemorySpace.VMEM),
            pl.BlockSpec(memory_space=pltpu.MemorySpace.VMEM),
        ],
        out_specs=pl.BlockSpec(memory_space=pltpu.MemorySpace.HBM),
        out_shape=jax.ShapeDtypeStruct((B, D_OUT), jnp.float32),
        scratch_shapes=[
            pltpu.VMEM((NX, CH, D_IN), jnp.float32),
            pltpu.VMEM((NO, CH, D_OUT), jnp.float32),
            pltpu.SemaphoreType.DMA((NX,)),
            pltpu.SemaphoreType.DMA((NO,)),
        ],
    )(image, W1, b1r, W2, b2r)


# speedup vs baseline: 1.0146x; 1.0146x over previous
import jax
import jax.numpy as jnp
from jax.experimental import pallas as pl
from jax.experimental.pallas import tpu as pltpu

B = 16384
D_IN = 1024
D_HID = 512
D_OUT = 512

CH = 1024
NCHUNK = B // CH
NBUF = 8


def _mlp_kernel(x_hbm, w1_ref, b1_ref, w2_ref, b2_ref, o_hbm,
                x_bufs, o_bufs, in_sems, out_sems):
    w1 = w1_ref[:]
    b1 = b1_ref[:]
    w2 = w2_ref[:]
    b2 = b2_ref[:]

    def in_copy(i, slot):
        return pltpu.make_async_copy(
            x_hbm.at[pl.ds(i * CH, CH), :], x_bufs.at[slot], in_sems.at[slot])

    def out_copy(i, slot):
        return pltpu.make_async_copy(
            o_bufs.at[slot], o_hbm.at[pl.ds(i * CH, CH), :], out_sems.at[slot])

    for k in range(NBUF):
        in_copy(k, k).start()

    for i in range(NCHUNK):
        slot = i % NBUF
        in_copy(i, slot).wait()

        h = jnp.dot(x_bufs[slot], w1, preferred_element_type=jnp.float32)
        h = jnp.maximum(h + b1, 0.0)
        o = jnp.dot(h, w2, preferred_element_type=jnp.float32) + b2

        if i >= NBUF:
            out_copy(i - NBUF, slot).wait()

        o_bufs[slot] = o
        out_copy(i, slot).start()

        if i + NBUF < NCHUNK:
            in_copy(i + NBUF, slot).start()

    for k in range(NBUF):
        i = NCHUNK - NBUF + k
        out_copy(i, i % NBUF).wait()


@jax.jit
def kernel(image, W1, b1, W2, b2):
    b1r = b1.reshape(1, D_HID)
    b2r = b2.reshape(1, D_OUT)
    return pl.pallas_call(
        _mlp_kernel,
        in_specs=[
            pl.BlockSpec(memory_space=pltpu.MemorySpace.HBM),
            pl.BlockSpec(memory_space=pltpu.MemorySpace.VMEM),
            pl.BlockSpec(memory_space=pltpu.MemorySpace.VMEM),
            pl.BlockSpec(memory_space=pltpu.MemorySpace.VMEM),
            pl.BlockSpec(memory_space=pltpu.MemorySpace.VMEM),
        ],
        out_specs=pl.BlockSpec(memory_space=pltpu.MemorySpace.HBM),
        out_shape=jax.ShapeDtypeStruct((B, D_OUT), jnp.float32),
        scratch_shapes=[
            pltpu.VMEM((NBUF, CH, D_IN), jnp.float32),
            pltpu.VMEM((NBUF, CH, D_OUT), jnp.float32),
            pltpu.SemaphoreType.DMA((NBUF,)),
            pltpu.SemaphoreType.DMA((NBUF,)),
        ],
    )(image, W1, b1r, W2, b2r)
